# Initial kernel scaffold; baseline (speedup 1.0000x reference)
import jax
import jax.numpy as jnp
from jax.experimental import pallas as pl


def kernel(x, proposalN):
    raise NotImplementedError("write your pallas kernel here")


# speedup vs baseline: 1.9842x; 1.9842x over previous
import functools

import jax
import jax.numpy as jnp
import numpy as np
from jax import lax
from jax.experimental import pallas as pl
from jax.experimental.pallas import tpu as pltpu
from jax.experimental.pallas import tpu_sc as plsc

_FM = 32
_RATIOS = [[8, 8], [6, 10], [10, 6], [12, 12], [10, 14], [14, 10], [16, 16], [14, 18], [18, 14]]
_GROUPS = [
    (0, (0, 1, 2), 1867, 0, 3),
    (1872, (3, 4, 5), 1315, 1867, 2),
    (3200, (6, 7, 8), 859, 3182, 1),
]
_TOTAL_PAD = 4064
_P2_COLS = 48
_P2_WORDS = 33 * _P2_COLS
_IOU_THRESH = 0.25
_NEG = float(np.float32(-np.inf))


def _build_consts():
    ia = np.full(_TOTAL_PAD, 47, np.int32)
    ib = np.full(_TOTAL_PAD, 47, np.int32)
    ic = np.full(_TOTAL_PAD, 47, np.int32)
    idd = np.full(_TOTAL_PAD, 47, np.int32)
    hw = np.ones(_TOTAL_PAD, np.float32)
    x0 = np.zeros(_TOTAL_PAD, np.float32)
    y0 = np.zeros(_TOTAL_PAD, np.float32)
    x1 = np.zeros(_TOTAL_PAD, np.float32)
    y1 = np.zeros(_TOTAL_PAD, np.float32)

    def pidx(r, c):
        return r * _P2_COLS + (47 if c == 0 else c - 1)

    for base, ridx, count, _, _ in _GROUPS:
        p = base
        for r in ridx:
            h, w = _RATIOS[r]
            for i in range(_FM - h + 1):
                for j in range(_FM - w + 1):
                    ia[p] = pidx(i + h, j + w)
                    ib[p] = pidx(i, j + w)
                    ic[p] = pidx(i + h, j)
                    idd[p] = pidx(i, j)
                    hw[p] = float(h * w)
                    x0[p] = i
                    y0[p] = j
                    x1[p] = i + h - 1
                    y1[p] = j + w - 1
                    p += 1
        assert p - base == count
    idxc = np.stack([ia, ib, ic, idd])
    fc = np.stack([hw, x0, y0, x1, y1])
    return idxc, fc


_IDXC, _FC = _build_consts()


def _sc_body(x_hbm, idx_hbm, fc_hbm, ws_hbm, pk_hbm, ps_hbm,
             xv, p2, sc, vl, idxv, fcv, pkv, psv):
    wid = lax.axis_index("c") * 16 + lax.axis_index("s")
    pltpu.sync_copy(x_hbm.at[wid], xv)
    pltpu.sync_copy(idx_hbm, idxv)
    pltpu.sync_copy(fc_hbm, fcv)

    iota = lax.broadcasted_iota(jnp.int32, (16,), 0)
    zf = jnp.zeros((16,), jnp.float32)
    zi = jnp.zeros((16,), jnp.int32)
    neg = jnp.full((16,), _NEG, jnp.float32)

    def zero_body(i, c):
        p2[pl.ds(i * 16, 16)] = zf
        return c
    lax.fori_loop(0, _P2_WORDS // 16, zero_body, 0)

    def row_body(i, c):
        r0 = xv[pl.ds(i * 32, 16)]
        r1 = xv[pl.ds(i * 32 + 16, 16)]
        c0 = plsc.cumsum(r0)
        c1 = plsc.cumsum(r1) + jnp.sum(r0)
        b = i * _P2_COLS
        p2[pl.ds(b + _P2_COLS, 16)] = p2[pl.ds(b, 16)] + c0
        p2[pl.ds(b + _P2_COLS + 16, 16)] = p2[pl.ds(b + 16, 16)] + c1
        return c
    lax.fori_loop(0, _FM, row_body, 0)

    def pool_body(v, c):
        o = v * 16
        pa = plsc.load_gather(p2, [idxv[0, pl.ds(o, 16)]])
        pb = plsc.load_gather(p2, [idxv[1, pl.ds(o, 16)]])
        pc = plsc.load_gather(p2, [idxv[2, pl.ds(o, 16)]])
        pd = plsc.load_gather(p2, [idxv[3, pl.ds(o, 16)]])
        sc[pl.ds(o, 16)] = (pa - pb - pc + pd) / fcv[0, pl.ds(o, 16)]
        pos = iota + o
        ok = ((pos < 1867)
              | ((pos >= 1872) & (pos < 1872 + 1315))
              | ((pos >= 3200) & (pos < 3200 + 859)))
        vl[pl.ds(o, 16)] = jnp.where(ok, 1, 0)
        return c
    lax.fori_loop(0, _TOTAL_PAD // 16, pool_body, 0)

    picks_vec = zi
    ps_vec = zf
    slot = 0
    for base, _, _, lo, npicks in _GROUPS:
        nv = 0
        for b2, _, cnt, _, _ in _GROUPS:
            if b2 == base:
                nv = -(-cnt // 16)
        last = jnp.int32(0)
        for t in range(npicks):
            def amax_body(i, carry, base=base):
                mvec, ivec, vm = carry
                o = base + i * 16
                s = sc[pl.ds(o, 16)]
                v = vl[pl.ds(o, 16)]
                smask = jnp.where(v > 0, s, neg)
                gt = smask > mvec
                mvec = jnp.where(gt, smask, mvec)
                ivec = jnp.where(gt, jnp.full((16,), i, jnp.int32), ivec)
                vm = jnp.maximum(vm, v)
                return mvec, ivec, vm
            mvec, ivec, vm = lax.fori_loop(0, nv, amax_body, (neg, zi, zi))
            m = jnp.max(mvec)
            anyv = jnp.max(vm) > 0
            cand = jnp.where(mvec == m, ivec * 16 + iota, jnp.full((16,), 2**30, jnp.int32))
            pick = jnp.where(anyv, jnp.min(cand), last)
            last = pick
            splat = jnp.full((16,), base + pick, jnp.int32)
            spv = plsc.load_gather(sc, [splat])
            slot_m = iota == slot
            picks_vec = jnp.where(slot_m, jnp.full((16,), pick + lo, jnp.int32), picks_vec)
            ps_vec = jnp.where(slot_m, spv, ps_vec)
            slot += 1
            if t < npicks - 1:
                one = jnp.full((16,), 1, jnp.int32)
                cx0 = plsc.load_gather(fcv, [one, splat])
                cy0 = plsc.load_gather(fcv, [one + 1, splat])
                cx1 = plsc.load_gather(fcv, [one + 2, splat])
                cy1 = plsc.load_gather(fcv, [one + 3, splat])
                car = plsc.load_gather(fcv, [jnp.zeros((16,), jnp.int32), splat])

                def sup_body(i, c, base=base, cx0=cx0, cy0=cy0, cx1=cx1, cy1=cy1, car=car):
                    o = base + i * 16
                    wx0 = fcv[1, pl.ds(o, 16)]
                    wy0 = fcv[2, pl.ds(o, 16)]
                    wx1 = fcv[3, pl.ds(o, 16)]
                    wy1 = fcv[4, pl.ds(o, 16)]
                    war = fcv[0, pl.ds(o, 16)]
                    v = vl[pl.ds(o, 16)]
                    lx = jnp.minimum(wx1, cx1) - jnp.maximum(wx0, cx0) + 1.0
                    ly = jnp.minimum(wy1, cy1) - jnp.maximum(wy0, cy0) + 1.0
                    inter = jnp.where((lx < 0.0) | (ly < 0.0), 0.0, lx * ly)
                    iou = inter / (war + car - inter)
                    vl[pl.ds(o, 16)] = jnp.where(iou <= _IOU_THRESH, v, 0)
                    return c
                lax.fori_loop(0, nv, sup_body, 0)

    pkv[...] = picks_vec
    psv[...] = ps_vec
    pltpu.sync_copy(sc, ws_hbm.at[wid])
    pltpu.sync_copy(pkv, pk_hbm.at[wid])
    pltpu.sync_copy(psv, ps_hbm.at[wid])


@jax.jit
def _launch(x2):
    mesh = plsc.VectorSubcoreMesh(core_axis_name="c", subcore_axis_name="s")
    f = functools.partial(
        pl.kernel,
        mesh=mesh,
        compiler_params=pltpu.CompilerParams(needs_layout_passes=False),
        out_type=[
            jax.ShapeDtypeStruct((32, _TOTAL_PAD), jnp.float32),
            jax.ShapeDtypeStruct((32, 16), jnp.int32),
            jax.ShapeDtypeStruct((32, 16), jnp.float32),
        ],
        scratch_types=[
            pltpu.VMEM((1024,), jnp.float32),
            pltpu.VMEM((_P2_WORDS,), jnp.float32),
            pltpu.VMEM((_TOTAL_PAD,), jnp.float32),
            pltpu.VMEM((_TOTAL_PAD,), jnp.int32),
            pltpu.VMEM((4, _TOTAL_PAD), jnp.int32),
            pltpu.VMEM((5, _TOTAL_PAD), jnp.float32),
            pltpu.VMEM((16,), jnp.int32),
            pltpu.VMEM((16,), jnp.float32),
        ],
    )(_sc_body)
    return f(x2, jnp.asarray(_IDXC), jnp.asarray(_FC))


def kernel(x, proposalN):
    x2 = x.reshape(32, 1024)
    ws, pk, ps = _launch(x2)
    window_scores = jnp.concatenate(
        [ws[:, b:b + n] for b, _, n, _, _ in _GROUPS], axis=1)
    indices = (pk[:, :6] + (proposalN - 6)).astype(jnp.int64)
    return (indices, ps[:, :6], window_scores)


# speedup vs baseline: 2.2175x; 1.1176x over previous
import functools

import jax
import jax.numpy as jnp
import numpy as np
from jax import lax
from jax.experimental import pallas as pl
from jax.experimental.pallas import tpu as pltpu
from jax.experimental.pallas import tpu_sc as plsc

_FM = 32
_RATIOS = [[8, 8], [6, 10], [10, 6], [12, 12], [10, 14], [14, 10], [16, 16], [14, 18], [18, 14]]
_GROUPS = [
    (0, (0, 1, 2), 1867, 0, 3),
    (1872, (3, 4, 5), 1315, 1867, 2),
    (3200, (6, 7, 8), 859, 3182, 1),
]
_TOTAL_PAD = 4064
_P2_COLS = 48
_P2_WORDS = 33 * _P2_COLS
_IOU_THRESH = 0.25
_NEG = float(np.float32(-np.inf))


def _build_consts():
    ia = np.full(_TOTAL_PAD, 47, np.int32)
    ib = np.full(_TOTAL_PAD, 47, np.int32)
    ic = np.full(_TOTAL_PAD, 47, np.int32)
    idd = np.full(_TOTAL_PAD, 47, np.int32)
    hw = np.ones(_TOTAL_PAD, np.float32)
    x0 = np.zeros(_TOTAL_PAD, np.float32)
    y0 = np.zeros(_TOTAL_PAD, np.float32)
    x1 = np.zeros(_TOTAL_PAD, np.float32)
    y1 = np.zeros(_TOTAL_PAD, np.float32)

    def pidx(r, c):
        return r * _P2_COLS + (47 if c == 0 else c - 1)

    for base, ridx, count, _, _ in _GROUPS:
        p = base
        for r in ridx:
            h, w = _RATIOS[r]
            for i in range(_FM - h + 1):
                for j in range(_FM - w + 1):
                    ia[p] = pidx(i + h, j + w)
                    ib[p] = pidx(i, j + w)
                    ic[p] = pidx(i + h, j)
                    idd[p] = pidx(i, j)
                    hw[p] = float(h * w)
                    x0[p] = i
                    y0[p] = j
                    x1[p] = i + h - 1
                    y1[p] = j + w - 1
                    p += 1
        assert p - base == count
    idxc = np.stack([ia, ib, ic, idd])
    fc = np.stack([hw, x0, y0, x1, y1])
    return idxc, fc


_IDXC, _FC = _build_consts()


def _sc_body(x_hbm, idx_hbm, fc_hbm, ws_hbm, pk_hbm, ps_hbm,
             xv, p2, sc, vl, idxv, fcv, pkv, psv):
    wid = lax.axis_index("c") * 16 + lax.axis_index("s")
    pltpu.sync_copy(x_hbm.at[wid], xv)
    pltpu.sync_copy(idx_hbm, idxv)
    pltpu.sync_copy(fc_hbm, fcv)

    iota = lax.broadcasted_iota(jnp.int32, (16,), 0)
    zf = jnp.zeros((16,), jnp.float32)
    zi = jnp.zeros((16,), jnp.int32)
    neg = jnp.full((16,), _NEG, jnp.float32)
    big = jnp.full((16,), 2**30, jnp.int32)

    p2[pl.ds(0, 16)] = zf
    p2[pl.ds(16, 16)] = zf
    p2[pl.ds(32, 16)] = zf

    def row_body(i, c):
        r0 = xv[pl.ds(i * 32, 16)]
        r1 = xv[pl.ds(i * 32 + 16, 16)]
        c0 = plsc.cumsum(r0)
        c1 = plsc.cumsum(r1) + jnp.sum(r0)
        b = i * _P2_COLS
        p2[pl.ds(b + _P2_COLS, 16)] = p2[pl.ds(b, 16)] + c0
        p2[pl.ds(b + _P2_COLS + 16, 16)] = p2[pl.ds(b + 16, 16)] + c1
        p2[pl.ds(b + _P2_COLS + 32, 16)] = zf
        return c
    lax.fori_loop(0, _FM, row_body, 0, unroll=2)

    def pool_body(v, carry):
        m0, i0, m1, i1, m2, i2 = carry
        o = v * 16
        pa = plsc.load_gather(p2, [idxv[0, pl.ds(o, 16)]])
        pb = plsc.load_gather(p2, [idxv[1, pl.ds(o, 16)]])
        pc = plsc.load_gather(p2, [idxv[2, pl.ds(o, 16)]])
        pd = plsc.load_gather(p2, [idxv[3, pl.ds(o, 16)]])
        s = (pa - pb - pc + pd) / fcv[0, pl.ds(o, 16)]
        sc[pl.ds(o, 16)] = s
        pos = iota + o
        ok = ((pos < 1867)
              | ((pos >= 1872) & (pos < 1872 + 1315))
              | ((pos >= 3200) & (pos < 3200 + 859)))
        vl[pl.ds(o, 16)] = jnp.where(ok, 1, 0)
        smask = jnp.where(ok, s, neg)
        vsp = jnp.full((16,), v, jnp.int32)
        isp = jnp.full((16,), v, jnp.int32)
        g0 = vsp < 117
        g1 = (vsp >= 117) & (vsp < 200)
        g2 = vsp >= 200
        gt0 = g0 & (smask > m0)
        m0 = jnp.where(gt0, smask, m0)
        i0 = jnp.where(gt0, isp, i0)
        gt1 = g1 & (smask > m1)
        m1 = jnp.where(gt1, smask, m1)
        i1 = jnp.where(gt1, isp, i1)
        gt2 = g2 & (smask > m2)
        m2 = jnp.where(gt2, smask, m2)
        i2 = jnp.where(gt2, isp, i2)
        return m0, i0, m1, i1, m2, i2
    m0, i0, m1, i1, m2, i2 = lax.fori_loop(
        0, _TOTAL_PAD // 16, pool_body, (neg, zi, neg, zi, neg, zi), unroll=2)

    first = {0: (m0, i0), 1872: (m1, i1), 3200: (m2, i2)}
    picks_vec = zi
    ps_vec = zf
    slot = 0
    for base, _, cnt, lo, npicks in _GROUPS:
        nv = -(-cnt // 16)
        mvec, ivec = first[base]
        m = jnp.max(mvec)
        cand = jnp.where(mvec == m, ivec * 16 + iota, big)
        pick = jnp.min(cand) - base
        last = pick
        for t in range(npicks):
            if t > 0:
                m = jnp.max(mvec)
                anyv = jnp.max(vm) > 0
                cand = jnp.where(mvec == m, ivec * 16 + iota, big)
                pick = jnp.where(anyv, jnp.min(cand), last)
                last = pick
            splat = jnp.full((16,), base + pick, jnp.int32)
            spv = plsc.load_gather(sc, [splat])
            slot_m = iota == slot
            picks_vec = jnp.where(slot_m, jnp.full((16,), pick + lo, jnp.int32), picks_vec)
            ps_vec = jnp.where(slot_m, spv, ps_vec)
            slot += 1
            if t < npicks - 1:
                one = jnp.full((16,), 1, jnp.int32)
                cx0 = plsc.load_gather(fcv, [one, splat])
                cy0 = plsc.load_gather(fcv, [one + 1, splat])
                cx1 = plsc.load_gather(fcv, [one + 2, splat])
                cy1 = plsc.load_gather(fcv, [one + 3, splat])
                car = plsc.load_gather(fcv, [jnp.zeros((16,), jnp.int32), splat])

                def fused_body(i, carry, base=base,
                               cx0=cx0, cy0=cy0, cx1=cx1, cy1=cy1, car=car):
                    mvec, ivec, vm = carry
                    o = base + i * 16
                    s = sc[pl.ds(o, 16)]
                    v = vl[pl.ds(o, 16)]
                    wx0 = fcv[1, pl.ds(o, 16)]
                    wy0 = fcv[2, pl.ds(o, 16)]
                    wx1 = fcv[3, pl.ds(o, 16)]
                    wy1 = fcv[4, pl.ds(o, 16)]
                    war = fcv[0, pl.ds(o, 16)]
                    lx = jnp.minimum(wx1, cx1) - jnp.maximum(wx0, cx0) + 1.0
                    ly = jnp.minimum(wy1, cy1) - jnp.maximum(wy0, cy0) + 1.0
                    inter = jnp.where((lx < 0.0) | (ly < 0.0), 0.0, lx * ly)
                    keep = inter <= _IOU_THRESH * (war + car - inter)
                    v = jnp.where(keep, v, 0)
                    vl[pl.ds(o, 16)] = v
                    smask = jnp.where(v > 0, s, neg)
                    gt = smask > mvec
                    mvec = jnp.where(gt, smask, mvec)
                    ivec = jnp.where(gt, jnp.full((16,), i, jnp.int32), ivec)
                    vm = jnp.maximum(vm, v)
                    return mvec, ivec, vm
                mvec, ivec, vm = lax.fori_loop(
                    0, nv, fused_body, (neg, zi, zi), unroll=2)

    pkv[...] = picks_vec
    psv[...] = ps_vec
    pltpu.sync_copy(sc, ws_hbm.at[wid])
    pltpu.sync_copy(pkv, pk_hbm.at[wid])
    pltpu.sync_copy(psv, ps_hbm.at[wid])


@jax.jit
def _launch(x2):
    mesh = plsc.VectorSubcoreMesh(core_axis_name="c", subcore_axis_name="s")
    f = functools.partial(
        pl.kernel,
        mesh=mesh,
        compiler_params=pltpu.CompilerParams(needs_layout_passes=False),
        out_type=[
            jax.ShapeDtypeStruct((32, _TOTAL_PAD), jnp.float32),
            jax.ShapeDtypeStruct((32, 16), jnp.int32),
            jax.ShapeDtypeStruct((32, 16), jnp.float32),
        ],
        scratch_types=[
            pltpu.VMEM((1024,), jnp.float32),
            pltpu.VMEM((_P2_WORDS,), jnp.float32),
            pltpu.VMEM((_TOTAL_PAD,), jnp.float32),
            pltpu.VMEM((_TOTAL_PAD,), jnp.int32),
            pltpu.VMEM((4, _TOTAL_PAD), jnp.int32),
            pltpu.VMEM((5, _TOTAL_PAD), jnp.float32),
            pltpu.VMEM((16,), jnp.int32),
            pltpu.VMEM((16,), jnp.float32),
        ],
    )(_sc_body)
    return f(x2, jnp.asarray(_IDXC), jnp.asarray(_FC))


def kernel(x, proposalN):
    x2 = x.reshape(32, 1024)
    ws, pk, ps = _launch(x2)
    window_scores = jnp.concatenate(
        [ws[:, b:b + n] for b, _, n, _, _ in _GROUPS], axis=1)
    indices = (pk[:, :6] + (proposalN - 6)).astype(jnp.int64)
    return (indices, ps[:, :6], window_scores)
